# baseline (speedup 1.0000x reference)
import jax
import jax.numpy as jnp
from jax.experimental import pallas as pl
from jax.experimental.pallas import tpu as pltpu
from jax.experimental.pallas import tpu_sc as plsc

_LANES = 16
_CHUNK_ROWS = 8
_SC_BATCHES = 1
_TC_ROWS = 512


def _sc_add(x2, pos_table, nb):
    (_, d) = x2.shape
    (m, _) = pos_table.shape
    m_chunks = m // _CHUNK_ROWS

    mesh = plsc.VectorSubcoreMesh(
        core_axis_name="core", subcore_axis_name="subcore"
    )

    @pl.kernel(out_type=jax.ShapeDtypeStruct((nb * m, d), x2.dtype), mesh=mesh)
    def k(x_hbm, pos_hbm, o_hbm):
        def body(*refs):
            xs = refs[:nb]
            pos_v = refs[nb]
            os_ = refs[nb + 1:]

            @plsc.parallel_loop(0, d, step=_LANES, unroll=2)
            def _(c):
                for r in range(_CHUNK_ROWS):
                    slc = (pl.ds(r, 1), pl.ds(c, _LANES))
                    p = pos_v.at[*slc][...]
                    for xi, oi in zip(xs, os_):
                        oi.at[*slc][...] = xi.at[*slc][...] + p

        blk = (_CHUNK_ROWS, d)
        x_specs = [
            pl.BlockSpec(block_shape=blk,
                         index_map=lambda i, bb=bb: (bb * m_chunks + i, 0))
            for bb in range(nb)
        ]
        pos_spec = pl.BlockSpec(block_shape=blk, index_map=lambda i: (i, 0))
        pltpu.emit_pipeline(
            body,
            grid=(m_chunks,),
            in_specs=x_specs + [pos_spec],
            out_specs=list(x_specs),
            core_axis_name=("core", "subcore"),
            dimension_semantics=(pltpu.PARALLEL,),
        )(*([x_hbm] * nb), pos_hbm, *([o_hbm] * nb))

    return k(x2, pos_table)


def _tc_add(x, pos_table, nb_skip):
    b, m, d = x.shape

    def body(x_ref, pos_ref, o_ref):
        o_ref[...] = x_ref[...] + pos_ref[...][None, :, :]

    return pl.pallas_call(
        body,
        grid=(m // _TC_ROWS, b - nb_skip),
        in_specs=[
            pl.BlockSpec((1, _TC_ROWS, d), lambda i, j: (j + nb_skip, i, 0)),
            pl.BlockSpec((_TC_ROWS, d), lambda i, j: (i, 0)),
        ],
        out_specs=pl.BlockSpec((1, _TC_ROWS, d), lambda i, j: (j + nb_skip, i, 0)),
        out_shape=jax.ShapeDtypeStruct((b, m, d), x.dtype),
    )(x, pos_table)


def kernel(x, pos_table):
    b, m, d = x.shape
    out_tc = _tc_add(x, pos_table, _SC_BATCHES)
    out_sc = _sc_add(x.reshape(b * m, d), pos_table, _SC_BATCHES)
    out_sc2 = jax.lax.optimization_barrier(out_sc.reshape(_SC_BATCHES, m, d))
    return jax.lax.dynamic_update_slice(out_tc, out_sc2, (0, 0, 0))


# speedup vs baseline: 1.1821x; 1.1821x over previous
import jax
import jax.numpy as jnp
from jax.experimental import pallas as pl
from jax.experimental.pallas import tpu as pltpu
from jax.experimental.pallas import tpu_sc as plsc

_LANES = 16
_CHUNK_ROWS = 16
_PAIR = 2


def _sc_add(x2, pos_table):
    (bm, d) = x2.shape
    (m, _) = pos_table.shape
    b = bm // m
    n_pairs = b // _PAIR
    m_chunks = m // _CHUNK_ROWS

    mesh = plsc.VectorSubcoreMesh(
        core_axis_name="core", subcore_axis_name="subcore"
    )

    @pl.kernel(out_type=jax.ShapeDtypeStruct((bm, d), x2.dtype), mesh=mesh)
    def k(x_hbm, pos_hbm, o_hbm):
        def body(*refs):
            xs = refs[:_PAIR]
            pos_v = refs[_PAIR]
            os_ = refs[_PAIR + 1:]

            @plsc.parallel_loop(0, d, step=_LANES, unroll=2)
            def _(c):
                for r in range(_CHUNK_ROWS):
                    slc = (pl.ds(r, 1), pl.ds(c, _LANES))
                    p = pos_v.at[*slc][...]
                    for xi, oi in zip(xs, os_):
                        oi.at[*slc][...] = xi.at[*slc][...] + p

        blk = (_CHUNK_ROWS, d)
        x_specs = [
            pl.BlockSpec(
                block_shape=blk,
                index_map=lambda i, j, bb=bb: ((j * _PAIR + bb) * m_chunks + i, 0),
            )
            for bb in range(_PAIR)
        ]
        pos_spec = pl.BlockSpec(block_shape=blk, index_map=lambda i, j: (i, 0))
        pltpu.emit_pipeline(
            body,
            grid=(m_chunks, n_pairs),
            in_specs=x_specs + [pos_spec],
            out_specs=list(x_specs),
            core_axis_name=("core", "subcore"),
            dimension_semantics=(pltpu.PARALLEL, pltpu.PARALLEL),
        )(*([x_hbm] * _PAIR), pos_hbm, *([o_hbm] * _PAIR))

    return k(x2, pos_table)


def kernel(x, pos_table):
    b, m, d = x.shape
    out2 = _sc_add(x.reshape(b * m, d), pos_table)
    return out2.reshape(b, m, d)
